# baseline (speedup 1.0000x reference)
import jax
import jax.numpy as jnp
from jax import lax
from jax.experimental import pallas as pl


def _ple_body(x_ref, lo_ref, hi_ref, o_ref):
    xv = x_ref[...]
    lo = lo_ref[...]
    hi = hi_ref[...]
    col = lax.broadcasted_iota(jnp.int32, lo.shape, 1)
    n_bins = lo.shape[1]
    left = (xv < lo) & (col > 0)
    right = (xv >= hi) & (col < (n_bins - 1))
    t = (xv - lo) / (hi - lo)
    enc = jnp.where(left, 0.0, 1.0)
    o_ref[...] = jnp.where(right == left, t, enc)


def kernel(x, bins):
    n = x.shape[0]
    nb = bins.shape[0]
    lo = bins.reshape(1, nb)
    hi = jnp.concatenate(
        [bins[1:], jnp.array([-1.0], dtype=bins.dtype)]).reshape(1, nb)
    block = 8000
    out = pl.pallas_call(
        _ple_body,
        grid=(n // block,),
        in_specs=[
            pl.BlockSpec((block, 1), lambda i: (i, 0)),
            pl.BlockSpec((1, nb), lambda i: (0, 0)),
            pl.BlockSpec((1, nb), lambda i: (0, 0)),
        ],
        out_specs=pl.BlockSpec((block, nb), lambda i: (i, 0)),
        out_shape=jax.ShapeDtypeStruct((n, nb), jnp.float32),
    )(x, lo, hi)
    return out.reshape(n, 1, nb)


# speedup vs baseline: 1.2345x; 1.2345x over previous
import jax
import jax.numpy as jnp
from jax import lax
from jax.experimental import pallas as pl

_XPACK = 8


def _ple_body(x_ref, sel_ref, lo_ref, hi_ref, g0_ref, l15_ref, o_ref):
    xv = x_ref[...]
    sel = sel_ref[...]
    xb = lax.dot_general(xv, sel, (((1,), (0,)), ((), ())),
                         precision=lax.Precision.HIGHEST,
                         preferred_element_type=jnp.float32)
    lo = lo_ref[...]
    hi = hi_ref[...]
    g0 = g0_ref[...] != 0
    l15 = l15_ref[...] != 0
    left = (xb < lo) & g0
    right = (xb >= hi) & l15
    t = (xb - lo) / (hi - lo)
    enc = jnp.where(left, 0.0, 1.0)
    o_ref[...] = jnp.where(right == left, t, enc)


def kernel(x, bins):
    n = x.shape[0]
    nb = bins.shape[0]
    lanes = _XPACK * nb
    x2 = x.reshape(n // _XPACK, _XPACK)
    rows = n // _XPACK

    lo1 = bins
    hi1 = jnp.concatenate([bins[1:], jnp.array([-1.0], dtype=bins.dtype)])
    lo = jnp.tile(lo1, _XPACK).reshape(1, lanes)
    hi = jnp.tile(hi1, _XPACK).reshape(1, lanes)
    j = jnp.arange(lanes, dtype=jnp.int32) % nb
    g0 = (j > 0).astype(jnp.int32).reshape(1, lanes)
    l15 = (j < nb - 1).astype(jnp.int32).reshape(1, lanes)
    sel = (jnp.arange(lanes, dtype=jnp.int32)[None, :] // nb
           == jnp.arange(_XPACK, dtype=jnp.int32)[:, None]
           ).astype(jnp.float32)

    blk = 10000
    rep = lambda i: (0, 0)
    out = pl.pallas_call(
        _ple_body,
        grid=(rows // blk,),
        in_specs=[
            pl.BlockSpec((blk, _XPACK), lambda i: (i, 0)),
            pl.BlockSpec((_XPACK, lanes), rep),
            pl.BlockSpec((1, lanes), rep),
            pl.BlockSpec((1, lanes), rep),
            pl.BlockSpec((1, lanes), rep),
            pl.BlockSpec((1, lanes), rep),
        ],
        out_specs=pl.BlockSpec((blk, lanes), lambda i: (i, 0)),
        out_shape=jax.ShapeDtypeStruct((rows, lanes), jnp.float32),
    )(x2, sel, lo, hi, g0, l15)
    return out.reshape(n, 1, nb)


# speedup vs baseline: 1.3272x; 1.0751x over previous
import jax
import jax.numpy as jnp
from jax import lax
from jax.experimental import pallas as pl

_XROW = 128
_LANES = 2048


def _ple_body(x_ref, sel_ref, lo_ref, hi_ref, g0_ref, l15_ref, o_ref):
    xv = x_ref[...]
    sel = sel_ref[...]
    xb = lax.dot_general(xv, sel, (((1,), (0,)), ((), ())),
                         precision=lax.Precision.HIGHEST,
                         preferred_element_type=jnp.float32)
    lo = lo_ref[...]
    hi = hi_ref[...]
    g0 = g0_ref[...] != 0
    l15 = l15_ref[...] != 0
    left = (xb < lo) & g0
    right = (xb >= hi) & l15
    t = (xb - lo) / (hi - lo)
    enc = jnp.where(left, 0.0, 1.0)
    o_ref[...] = jnp.where(right == left, t, enc)


def kernel(x, bins):
    n = x.shape[0]
    nb = bins.shape[0]
    rows = n // _XROW
    x3 = x.reshape(rows, _XROW)

    lo1 = bins
    hi1 = jnp.concatenate([bins[1:], jnp.array([-1.0], dtype=bins.dtype)])
    reps = _LANES // nb
    lo = jnp.tile(lo1, reps).reshape(1, _LANES)
    hi = jnp.tile(hi1, reps).reshape(1, _LANES)
    j = jnp.arange(_LANES, dtype=jnp.int32) % nb
    g0 = (j > 0).astype(jnp.int32).reshape(1, _LANES)
    l15 = (j < nb - 1).astype(jnp.int32).reshape(1, _LANES)
    L = jnp.arange(_LANES, dtype=jnp.int32)
    slot = 8 * (L // 128) + (L % 128) // nb
    sel = (slot[None, :] == jnp.arange(_XROW, dtype=jnp.int32)[:, None]
           ).astype(jnp.float32)

    blk = 1000
    grid = (rows + blk - 1) // blk
    rep = lambda i: (0, 0)
    out = pl.pallas_call(
        _ple_body,
        grid=(grid,),
        in_specs=[
            pl.BlockSpec((blk, _XROW), lambda i: (i, 0)),
            pl.BlockSpec((_XROW, _LANES), rep),
            pl.BlockSpec((1, _LANES), rep),
            pl.BlockSpec((1, _LANES), rep),
            pl.BlockSpec((1, _LANES), rep),
            pl.BlockSpec((1, _LANES), rep),
        ],
        out_specs=pl.BlockSpec((blk, _LANES), lambda i: (i, 0)),
        out_shape=jax.ShapeDtypeStruct((rows, _LANES), jnp.float32),
    )(x3, sel, lo, hi, g0, l15)
    return out.reshape(n, 1, nb)


# speedup vs baseline: 11.6377x; 8.7684x over previous
import jax
import jax.numpy as jnp
from jax.experimental import pallas as pl

_XROW = 128


def _ple_body(x_ref, a_ref, b_ref, lb_ref, ub_ref, o_ref):
    xv = x_ref[...]
    xb = jnp.broadcast_to(xv[None, :, None, :], (2,) + xv.shape[:1] + (8, 128))
    a = a_ref[...]
    b = b_ref[...]
    lb = lb_ref[...]
    ub = ub_ref[...]
    t = (xb - b) * a
    o_ref[...] = jnp.minimum(jnp.maximum(t, lb), ub)


def kernel(x, bins):
    n = x.shape[0]
    nb = bins.shape[0]
    rows = n // _XROW
    x3 = x.reshape(rows, _XROW)

    lo = bins
    hi = jnp.concatenate([bins[1:], jnp.array([-1.0], dtype=bins.dtype)])
    inv = 1.0 / (hi - lo)
    neg = jnp.float32(-3.0e38)
    lbv = jnp.where((jnp.arange(nb) == 0) | (jnp.arange(nb) == nb - 1), neg, 0.0)
    ubv = jnp.where(jnp.arange(nb) == nb - 1, 0.0, 1.0).astype(jnp.float32)

    def sub(v):
        return jnp.broadcast_to(
            v.astype(jnp.float32).reshape(2, 1, 8, 1), (2, 1, 8, _XROW))

    a4, b4, lb4, ub4 = sub(inv), sub(lo), sub(lbv), sub(ubv)

    blk = 1000
    grid = (rows + blk - 1) // blk
    rep = lambda i: (0, 0, 0, 0)
    out = pl.pallas_call(
        _ple_body,
        grid=(grid,),
        in_specs=[
            pl.BlockSpec((blk, _XROW), lambda i: (i, 0)),
            pl.BlockSpec((2, 1, 8, _XROW), rep),
            pl.BlockSpec((2, 1, 8, _XROW), rep),
            pl.BlockSpec((2, 1, 8, _XROW), rep),
            pl.BlockSpec((2, 1, 8, _XROW), rep),
        ],
        out_specs=pl.BlockSpec((2, blk, 8, _XROW), lambda i: (0, i, 0, 0)),
        out_shape=jax.ShapeDtypeStruct((2, rows, 8, _XROW), jnp.float32),
    )(x3, a4, b4, lb4, ub4)
    return out.transpose(1, 3, 0, 2).reshape(n, nb)[:, None, :]
